# baseline (speedup 1.0000x reference)
import functools

import jax
import jax.numpy as jnp
from jax import lax
from jax.experimental import pallas as pl
from jax.experimental.pallas import tpu as pltpu
from jax.experimental.pallas import tpu_sc as plsc

RADIUS = 0.2
MAXK = 64
NEG = -1e9


def _y_body(x_ref, w_ref, b_ref, y_ref):
    y_ref[...] = (
        jnp.dot(x_ref[...], w_ref[...], preferred_element_type=jnp.float32)
        + b_ref[...]
    ).astype(jnp.bfloat16)


def _stage1(x, w1a, b1):
    n, d = x.shape
    blk = 1024
    return pl.pallas_call(
        _y_body,
        grid=(n // blk,),
        in_specs=[
            pl.BlockSpec((blk, d), lambda i: (i, 0)),
            pl.BlockSpec((d, d), lambda i: (0, 0)),
            pl.BlockSpec((1, d), lambda i: (0, 0)),
        ],
        out_specs=pl.BlockSpec((blk, d), lambda i: (i, 0)),
        out_shape=jax.ShapeDtypeStruct((n, d), jnp.bfloat16),
    )(x, w1a, b1.reshape(1, d))


NBUF = 4


def _stage2(posx, posy, posz, batch, y, coff, nch_):
    n = posx.shape[0]
    d = y.shape[1] * 2
    nc = nch_
    k = MAXK
    r2 = RADIUS * RADIUS
    info = plsc.get_sparse_core_info()
    ncores, nsub = info.num_cores, info.num_subcores
    nw = ncores * nsub
    cpw = nc // nw
    mesh = plsc.VectorSubcoreMesh(core_axis_name="c", subcore_axis_name="s")

    scratch = [
        pltpu.VMEM((n,), jnp.float32),
        pltpu.VMEM((n,), jnp.float32),
        pltpu.VMEM((n,), jnp.float32),
        pltpu.VMEM((n + 16,), jnp.float32),
        pltpu.VMEM((n + 16,), jnp.int32),
    ]
    scratch += [pltpu.VMEM((k,), jnp.int32) for _ in range(NBUF)]
    scratch += [pltpu.VMEM((k, 4), jnp.float32) for _ in range(NBUF)]
    scratch += [pltpu.VMEM((k, d // 2), jnp.int32) for _ in range(NBUF)]
    scratch += [pltpu.SemaphoreType.DMA for _ in range(3 * NBUF)]

    @functools.partial(
        pl.kernel,
        mesh=mesh,
        out_type=[
            jax.ShapeDtypeStruct((nc * k, d // 2), jnp.int32),
            jax.ShapeDtypeStruct((nc * k, 4), jnp.float32),
        ],
        scratch_types=scratch,
        compiler_params=pltpu.CompilerParams(needs_layout_passes=False,
                                             use_tc_tiling_on_sc=False),
    )
    def sc(px_h, py_h, pz_h, b_h, y_h, g_h, rel_h,
           px_v, py_v, pz_v, dbuf, ibuf, *bufs):
        idxs = bufs[0:NBUF]
        rels = bufs[NBUF:2 * NBUF]
        rows = bufs[2 * NBUF:3 * NBUF]
        gsem = bufs[3 * NBUF:4 * NBUF]
        wg = bufs[4 * NBUF:5 * NBUF]
        wr = bufs[5 * NBUF:6 * NBUF]

        wid = lax.axis_index("s") * ncores + lax.axis_index("c")
        pltpu.sync_copy(px_h, px_v)
        pltpu.sync_copy(py_h, py_v)
        pltpu.sync_copy(pz_h, pz_v)
        pltpu.sync_copy(b_h, ibuf.at[pl.ds(0, n)])

        iota = lax.iota(jnp.int32, 16)
        lane0 = iota == 0
        big = jnp.int32(2 ** 30)

        def lower_bound(v):
            lo_, hi_ = 0, n
            for _ in range(14):
                mid = (lo_ + hi_) >> 1
                bm = jnp.max(plsc.load_gather(ibuf, [jnp.broadcast_to(mid, (16,))]))
                pred = bm < v
                lo_ = jnp.where(pred, mid + 1, lo_)
                hi_ = jnp.where(pred, hi_, mid)
            return lo_
        s1 = lower_bound(1)
        s2 = lower_bound(2)
        s3 = lower_bound(3)

        def select(cl, idx_v, rel_v):
            c = coff + wid * cpw + cl
            p = 2 * c
            pvec = jnp.broadcast_to(p, (16,))
            cxs = plsc.load_gather(px_v, [pvec])
            cys = plsc.load_gather(py_v, [pvec])
            czs = plsc.load_gather(pz_v, [pvec])
            bc = ((p >= s1).astype(jnp.int32) + (p >= s2).astype(jnp.int32)
                  + (p >= s3).astype(jnp.int32))
            lo = jnp.where(bc == 0, 0, jnp.where(bc == 1, s1,
                                                 jnp.where(bc == 2, s2, s3)))
            hi = jnp.where(bc == 0, s1, jnp.where(bc == 1, s2,
                                                  jnp.where(bc == 2, s3, n)))
            lo16 = lo >> 4
            hi16 = (hi + 15) >> 4

            @plsc.parallel_loop(lo16, hi16, unroll=8, carry=jnp.int32(0))
            def cnt0(i, cnt):
                base = i * 16
                gi = base + iota
                px = px_v[pl.ds(base, 16)]
                py = py_v[pl.ds(base, 16)]
                pz = pz_v[pl.ds(base, 16)]
                dx = px - cxs
                dy = py - cys
                dz = pz - czs
                d2 = dx * dx + dy * dy + dz * dz
                m = (gi >= lo) & (gi < hi) & (d2 <= r2)
                csum = plsc.cumsum(m.astype(jnp.int32))
                posn = csum + (cnt - 1)
                plsc.store_scatter(dbuf, [posn], d2, mask=m)
                plsc.store_scatter(ibuf, [posn], gi, mask=m)
                return cnt + csum[15]
            nch = (cnt0 + 15) >> 4

            def trim_body(cnt):
                def mx_body(i, m_):
                    base = i * 16
                    dv = dbuf[pl.ds(base, 16)]
                    inr = (base + iota) < cnt0
                    return jnp.maximum(m_, jnp.max(jnp.where(inr, dv, -1.0)))
                vmax = lax.fori_loop(0, nch, mx_body, jnp.float32(-1.0))

                def fp_body(i, pmin):
                    base = i * 16
                    dv = dbuf[pl.ds(base, 16)]
                    inr = (base + iota) < cnt0
                    eq = inr & (dv == vmax)
                    cand = jnp.where(eq, base + iota, big)
                    return jnp.minimum(pmin, jnp.min(cand))
                pmin = lax.fori_loop(0, nch, fp_body, big)
                plsc.store_scatter(
                    dbuf, [jnp.broadcast_to(pmin, (16,))],
                    jnp.broadcast_to(jnp.float32(-1.0), (16,)), mask=lane0)
                return cnt - 1
            cnt_sel = lax.while_loop(lambda cnt: cnt > k, trim_body, cnt0)

            for g in range(k // 16):
                idx_v[pl.ds(g * 16, 16)] = jnp.zeros((16,), jnp.int32)

            @plsc.parallel_loop(0, nch, unroll=4, carry=jnp.int32(0))
            def _cp(i, acc):
                base = i * 16
                dv = dbuf[pl.ds(base, 16)]
                inr = (base + iota) < cnt0
                m = inr & (dv >= 0.0)
                csum = plsc.cumsum(m.astype(jnp.int32))
                posn = csum + (acc - 1)
                iv = ibuf[pl.ds(base, 16)]
                plsc.store_scatter(idx_v, [posn], iv, mask=m & (posn < k))
                return acc + csum[15]

            for g in range(k // 16):
                slot = g * 16 + iota
                jv = idx_v[pl.ds(g * 16, 16)]
                gx = plsc.load_gather(px_v, [jv])
                gy = plsc.load_gather(py_v, [jv])
                gz = plsc.load_gather(pz_v, [jv])
                val = jnp.where(slot < cnt_sel, jnp.float32(1.0),
                                jnp.float32(0.0))
                plsc.store_scatter(rel_v, [slot, jnp.broadcast_to(0, (16,))],
                                   gx - cxs)
                plsc.store_scatter(rel_v, [slot, jnp.broadcast_to(1, (16,))],
                                   gy - cys)
                plsc.store_scatter(rel_v, [slot, jnp.broadcast_to(2, (16,))],
                                   gz - czs)
                plsc.store_scatter(rel_v, [slot, jnp.broadcast_to(3, (16,))],
                                   val)

        def out_slice(ref, cl):
            return ref.at[pl.ds((wid * cpw + cl) * k, k)]

        def ring_body(grp, _):
            for ph in range(NBUF):
                cl = grp * NBUF + ph
                j = ph
                jw = (ph + 1) % NBUF

                @pl.when(cl >= NBUF)
                def _():
                    pltpu.make_async_copy(rows[j], out_slice(g_h, cl - NBUF),
                                          wg[j]).wait()
                    pltpu.make_async_copy(rels[j], out_slice(rel_h, cl - NBUF),
                                          wr[j]).wait()

                select(cl, idxs[j], rels[j])
                pltpu.async_copy(y_h.at[idxs[j]], rows[j], gsem[j])

                @pl.when(cl >= NBUF - 1)
                def _():
                    pltpu.make_async_copy(y_h.at[idxs[jw]], rows[jw],
                                          gsem[jw]).wait()
                    pltpu.async_copy(rows[jw], out_slice(g_h, cl - (NBUF - 1)),
                                     wg[jw])
                    pltpu.async_copy(rels[jw], out_slice(rel_h, cl - (NBUF - 1)),
                                     wr[jw])
            return 0
        lax.fori_loop(0, cpw // NBUF, ring_body, 0)

        pltpu.make_async_copy(rows[0], out_slice(g_h, cpw - NBUF),
                              wg[0]).wait()
        pltpu.make_async_copy(rels[0], out_slice(rel_h, cpw - NBUF),
                              wr[0]).wait()
        for t in range(cpw - (NBUF - 1), cpw):
            j = t % NBUF
            pltpu.make_async_copy(y_h.at[idxs[j]], rows[j], gsem[j]).wait()
            pltpu.sync_copy(rows[j], out_slice(g_h, t))
            pltpu.sync_copy(rels[j], out_slice(rel_h, t))

    return sc(posx, posy, posz, batch, y)


def _mlp_body(bc, k, g_ref, rel_ref, w1b_ref, w2_ref, b2_ref, w3_ref,
              b3_ref, out_ref):
    g = g_ref[...].astype(jnp.float32)
    rel = rel_ref[...]
    h1 = jnp.maximum(
        g + jnp.dot(rel, w1b_ref[...], preferred_element_type=jnp.float32),
        0.0)
    h2 = jnp.maximum(
        jnp.dot(h1, w2_ref[...], preferred_element_type=jnp.float32)
        + b2_ref[...], 0.0)
    h3 = (jnp.dot(h2, w3_ref[...], preferred_element_type=jnp.float32)
          + b3_ref[...])
    no = h3.shape[-1]
    v3 = rel[:, 3:4].reshape(bc, k, 1)
    h3 = jnp.where(v3 > 0.5, h3.reshape(bc, k, no), NEG)
    mx = jnp.max(h3, axis=1)
    anyv = jnp.max(v3, axis=1)
    out_ref[...] = jnp.where(anyv > 0.5, mx, 0.0)


def _stage3(g, rel, w1b4, w2, b2, w3, b3):
    rows, d = g.shape
    k = MAXK
    nc = rows // k
    no = w3.shape[1]
    bc = 32
    body = functools.partial(_mlp_body, bc, k)
    return pl.pallas_call(
        body,
        grid=(nc // bc,),
        in_specs=[
            pl.BlockSpec((bc * k, d), lambda i: (i, 0)),
            pl.BlockSpec((bc * k, 4), lambda i: (i, 0)),
            pl.BlockSpec((4, d), lambda i: (0, 0)),
            pl.BlockSpec((d, d), lambda i: (0, 0)),
            pl.BlockSpec((1, d), lambda i: (0, 0)),
            pl.BlockSpec((d, no), lambda i: (0, 0)),
            pl.BlockSpec((1, no), lambda i: (0, 0)),
        ],
        out_specs=pl.BlockSpec((bc, no), lambda i: (i, 0)),
        out_shape=jax.ShapeDtypeStruct((nc, no), jnp.float32),
    )(g, rel, w1b4, w2, b2.reshape(1, d), w3, b3.reshape(1, no))


def kernel(x, pos, batch, W1, b1, W2, b2, W3, b3):
    n, d = x.shape
    w1a = W1[:d]
    w1b4 = jnp.concatenate(
        [W1[d:], jnp.zeros((1, W1.shape[1]), jnp.float32)], axis=0)
    y = _stage1(x, w1a, b1)
    y32 = jax.lax.bitcast_convert_type(y.reshape(n, d // 2, 2), jnp.int32)
    posx = pos[:, 0]
    posy = pos[:, 1]
    posz = pos[:, 2]
    nc = n // 2
    outs = []
    for half in range(2):
        g32, rel = _stage2(posx, posy, posz, batch, y32,
                           half * (nc // 2), nc // 2)
        g = jax.lax.bitcast_convert_type(g32, jnp.bfloat16).reshape(
            g32.shape[0], 2 * g32.shape[1])
        outs.append(_stage3(g, rel, w1b4, W2, b2, W3, b3))
    out = jnp.concatenate(outs, axis=0)
    idx = jnp.arange(0, n, 2)
    return (out, pos[idx], batch[idx])


# speedup vs baseline: 1.0160x; 1.0160x over previous
import functools

import jax
import jax.numpy as jnp
from jax import lax
from jax.experimental import pallas as pl
from jax.experimental.pallas import tpu as pltpu
from jax.experimental.pallas import tpu_sc as plsc

RADIUS = 0.2
MAXK = 64
NEG = -1e9


def _y_body(x_ref, w_ref, b_ref, y_ref):
    y_ref[...] = (
        jnp.dot(x_ref[...], w_ref[...], preferred_element_type=jnp.float32)
        + b_ref[...]
    ).astype(jnp.bfloat16)


def _stage1(x, w1a, b1):
    n, d = x.shape
    blk = 1024
    return pl.pallas_call(
        _y_body,
        grid=(n // blk,),
        in_specs=[
            pl.BlockSpec((blk, d), lambda i: (i, 0)),
            pl.BlockSpec((d, d), lambda i: (0, 0)),
            pl.BlockSpec((1, d), lambda i: (0, 0)),
        ],
        out_specs=pl.BlockSpec((blk, d), lambda i: (i, 0)),
        out_shape=jax.ShapeDtypeStruct((n, d), jnp.bfloat16),
    )(x, w1a, b1.reshape(1, d))


NBUF = 4


def _stage2(posx, posy, posz, batch, y):
    n = posx.shape[0]
    d = y.shape[1] * 2
    nc = n // 2
    k = MAXK
    r2 = RADIUS * RADIUS
    info = plsc.get_sparse_core_info()
    ncores, nsub = info.num_cores, info.num_subcores
    nw = ncores * nsub
    cpw = nc // nw
    mesh = plsc.VectorSubcoreMesh(core_axis_name="c", subcore_axis_name="s")

    scratch = [
        pltpu.VMEM((n,), jnp.float32),
        pltpu.VMEM((n,), jnp.float32),
        pltpu.VMEM((n,), jnp.float32),
        pltpu.VMEM((n + 16,), jnp.float32),
        pltpu.VMEM((n + 16,), jnp.int32),
    ]
    scratch += [pltpu.VMEM((k,), jnp.int32) for _ in range(NBUF)]
    scratch += [pltpu.VMEM((k, 4), jnp.float32) for _ in range(NBUF)]
    scratch += [pltpu.VMEM((k, d // 2), jnp.int32) for _ in range(NBUF)]
    scratch += [pltpu.SemaphoreType.DMA for _ in range(3 * NBUF)]

    @functools.partial(
        pl.kernel,
        mesh=mesh,
        out_type=[
            jax.ShapeDtypeStruct((nc * k, d // 2), jnp.int32),
            jax.ShapeDtypeStruct((nc * k, 4), jnp.float32),
        ],
        scratch_types=scratch,
        compiler_params=pltpu.CompilerParams(needs_layout_passes=False,
                                             use_tc_tiling_on_sc=False),
    )
    def sc(px_h, py_h, pz_h, b_h, y_h, g_h, rel_h,
           px_v, py_v, pz_v, dbuf, ibuf, *bufs):
        idxs = bufs[0:NBUF]
        rels = bufs[NBUF:2 * NBUF]
        rows = bufs[2 * NBUF:3 * NBUF]
        gsem = bufs[3 * NBUF:4 * NBUF]
        wg = bufs[4 * NBUF:5 * NBUF]
        wr = bufs[5 * NBUF:6 * NBUF]

        wid = lax.axis_index("s") * ncores + lax.axis_index("c")
        pltpu.sync_copy(px_h, px_v)
        pltpu.sync_copy(py_h, py_v)
        pltpu.sync_copy(pz_h, pz_v)
        pltpu.sync_copy(b_h, ibuf.at[pl.ds(0, n)])

        iota = lax.iota(jnp.int32, 16)
        lane0 = iota == 0
        big = jnp.int32(2 ** 30)

        def lower_bound(v):
            lo_, hi_ = 0, n
            for _ in range(14):
                mid = (lo_ + hi_) >> 1
                bm = jnp.max(plsc.load_gather(ibuf, [jnp.broadcast_to(mid, (16,))]))
                pred = bm < v
                lo_ = jnp.where(pred, mid + 1, lo_)
                hi_ = jnp.where(pred, hi_, mid)
            return lo_
        s1 = lower_bound(1)
        s2 = lower_bound(2)
        s3 = lower_bound(3)

        def select(cl, idx_v, rel_v):
            c = wid * cpw + cl
            p = 2 * c
            pvec = jnp.broadcast_to(p, (16,))
            cxs = plsc.load_gather(px_v, [pvec])
            cys = plsc.load_gather(py_v, [pvec])
            czs = plsc.load_gather(pz_v, [pvec])
            bc = ((p >= s1).astype(jnp.int32) + (p >= s2).astype(jnp.int32)
                  + (p >= s3).astype(jnp.int32))
            lo = jnp.where(bc == 0, 0, jnp.where(bc == 1, s1,
                                                 jnp.where(bc == 2, s2, s3)))
            hi = jnp.where(bc == 0, s1, jnp.where(bc == 1, s2,
                                                  jnp.where(bc == 2, s3, n)))
            lo16 = lo >> 4
            hi16 = (hi + 15) >> 4

            @plsc.parallel_loop(lo16, hi16, unroll=8, carry=jnp.int32(0))
            def cnt0(i, cnt):
                base = i * 16
                gi = base + iota
                px = px_v[pl.ds(base, 16)]
                py = py_v[pl.ds(base, 16)]
                pz = pz_v[pl.ds(base, 16)]
                dx = px - cxs
                dy = py - cys
                dz = pz - czs
                d2 = dx * dx + dy * dy + dz * dz
                m = (gi >= lo) & (gi < hi) & (d2 <= r2)
                csum = plsc.cumsum(m.astype(jnp.int32))
                posn = csum + (cnt - 1)
                plsc.store_scatter(dbuf, [posn], d2, mask=m)
                plsc.store_scatter(ibuf, [posn], gi, mask=m)
                return cnt + csum[15]
            nch = (cnt0 + 15) >> 4

            def trim_body(cnt):
                def mx_body(i, m_):
                    base = i * 16
                    dv = dbuf[pl.ds(base, 16)]
                    inr = (base + iota) < cnt0
                    return jnp.maximum(m_, jnp.max(jnp.where(inr, dv, -1.0)))
                vmax = lax.fori_loop(0, nch, mx_body, jnp.float32(-1.0))

                def fp_body(i, pmin):
                    base = i * 16
                    dv = dbuf[pl.ds(base, 16)]
                    inr = (base + iota) < cnt0
                    eq = inr & (dv == vmax)
                    cand = jnp.where(eq, base + iota, big)
                    return jnp.minimum(pmin, jnp.min(cand))
                pmin = lax.fori_loop(0, nch, fp_body, big)
                plsc.store_scatter(
                    dbuf, [jnp.broadcast_to(pmin, (16,))],
                    jnp.broadcast_to(jnp.float32(-1.0), (16,)), mask=lane0)
                return cnt - 1
            cnt_sel = lax.while_loop(lambda cnt: cnt > k, trim_body, cnt0)

            for g in range(k // 16):
                idx_v[pl.ds(g * 16, 16)] = jnp.zeros((16,), jnp.int32)

            @plsc.parallel_loop(0, nch, unroll=4, carry=jnp.int32(0))
            def _cp(i, acc):
                base = i * 16
                dv = dbuf[pl.ds(base, 16)]
                inr = (base + iota) < cnt0
                m = inr & (dv >= 0.0)
                csum = plsc.cumsum(m.astype(jnp.int32))
                posn = csum + (acc - 1)
                iv = ibuf[pl.ds(base, 16)]
                plsc.store_scatter(idx_v, [posn], iv, mask=m & (posn < k))
                return acc + csum[15]

            for g in range(k // 16):
                slot = g * 16 + iota
                jv = idx_v[pl.ds(g * 16, 16)]
                gx = plsc.load_gather(px_v, [jv])
                gy = plsc.load_gather(py_v, [jv])
                gz = plsc.load_gather(pz_v, [jv])
                val = jnp.where(slot < cnt_sel, jnp.float32(1.0),
                                jnp.float32(0.0))
                plsc.store_scatter(rel_v, [slot, jnp.broadcast_to(0, (16,))],
                                   gx - cxs)
                plsc.store_scatter(rel_v, [slot, jnp.broadcast_to(1, (16,))],
                                   gy - cys)
                plsc.store_scatter(rel_v, [slot, jnp.broadcast_to(2, (16,))],
                                   gz - czs)
                plsc.store_scatter(rel_v, [slot, jnp.broadcast_to(3, (16,))],
                                   val)

        def out_slice(ref, cl):
            return ref.at[pl.ds((wid * cpw + cl) * k, k)]

        def ring_body(grp, _):
            for ph in range(NBUF):
                cl = grp * NBUF + ph
                j = ph
                jw = (ph + 1) % NBUF

                @pl.when(cl >= NBUF)
                def _():
                    pltpu.make_async_copy(rows[j], out_slice(g_h, cl - NBUF),
                                          wg[j]).wait()
                    pltpu.make_async_copy(rels[j], out_slice(rel_h, cl - NBUF),
                                          wr[j]).wait()

                select(cl, idxs[j], rels[j])
                pltpu.async_copy(y_h.at[idxs[j]], rows[j], gsem[j])

                @pl.when(cl >= NBUF - 1)
                def _():
                    pltpu.make_async_copy(y_h.at[idxs[jw]], rows[jw],
                                          gsem[jw]).wait()
                    pltpu.async_copy(rows[jw], out_slice(g_h, cl - (NBUF - 1)),
                                     wg[jw])
                    pltpu.async_copy(rels[jw], out_slice(rel_h, cl - (NBUF - 1)),
                                     wr[jw])
            return 0
        lax.fori_loop(0, cpw // NBUF, ring_body, 0)

        pltpu.make_async_copy(rows[0], out_slice(g_h, cpw - NBUF),
                              wg[0]).wait()
        pltpu.make_async_copy(rels[0], out_slice(rel_h, cpw - NBUF),
                              wr[0]).wait()
        for t in range(cpw - (NBUF - 1), cpw):
            j = t % NBUF
            pltpu.make_async_copy(y_h.at[idxs[j]], rows[j], gsem[j]).wait()
            pltpu.sync_copy(rows[j], out_slice(g_h, t))
            pltpu.sync_copy(rels[j], out_slice(rel_h, t))

    return sc(posx, posy, posz, batch, y)


def _mlp_body(bc, k, g_ref, rel_ref, w1b_ref, w2_ref, b2_ref, w3_ref,
              b3_ref, out_ref):
    g = g_ref[...].astype(jnp.float32)
    rel = rel_ref[...]
    h1 = jnp.maximum(
        g + jnp.dot(rel, w1b_ref[...], preferred_element_type=jnp.float32),
        0.0)
    h2 = jnp.maximum(
        jnp.dot(h1, w2_ref[...], preferred_element_type=jnp.float32)
        + b2_ref[...], 0.0)
    h3 = (jnp.dot(h2, w3_ref[...], preferred_element_type=jnp.float32)
          + b3_ref[...])
    no = h3.shape[-1]
    v3 = rel[:, 3:4].reshape(bc, k, 1)
    h3 = jnp.where(v3 > 0.5, h3.reshape(bc, k, no), NEG)
    mx = jnp.max(h3, axis=1)
    anyv = jnp.max(v3, axis=1)
    out_ref[...] = jnp.where(anyv > 0.5, mx, 0.0)


def _stage3(g, rel, w1b4, w2, b2, w3, b3):
    rows, d = g.shape
    k = MAXK
    nc = rows // k
    no = w3.shape[1]
    bc = 32
    body = functools.partial(_mlp_body, bc, k)
    return pl.pallas_call(
        body,
        grid=(nc // bc,),
        in_specs=[
            pl.BlockSpec((bc * k, d), lambda i: (i, 0)),
            pl.BlockSpec((bc * k, 4), lambda i: (i, 0)),
            pl.BlockSpec((4, d), lambda i: (0, 0)),
            pl.BlockSpec((d, d), lambda i: (0, 0)),
            pl.BlockSpec((1, d), lambda i: (0, 0)),
            pl.BlockSpec((d, no), lambda i: (0, 0)),
            pl.BlockSpec((1, no), lambda i: (0, 0)),
        ],
        out_specs=pl.BlockSpec((bc, no), lambda i: (i, 0)),
        out_shape=jax.ShapeDtypeStruct((nc, no), jnp.float32),
    )(g, rel, w1b4, w2, b2.reshape(1, d), w3, b3.reshape(1, no))


def kernel(x, pos, batch, W1, b1, W2, b2, W3, b3):
    n, d = x.shape
    w1a = W1[:d]
    w1b4 = jnp.concatenate(
        [W1[d:], jnp.zeros((1, W1.shape[1]), jnp.float32)], axis=0)
    y = _stage1(x, w1a, b1)
    y32 = jax.lax.bitcast_convert_type(y.reshape(n, d // 2, 2), jnp.int32)
    posx = pos[:, 0]
    posy = pos[:, 1]
    posz = pos[:, 2]
    g32, rel = _stage2(posx, posy, posz, batch, y32)
    g = jax.lax.bitcast_convert_type(g32, jnp.bfloat16).reshape(
        g32.shape[0], 2 * g32.shape[1])
    out = _stage3(g, rel, w1b4, W2, b2, W3, b3)
    idx = jnp.arange(0, n, 2)
    return (out, pos[idx], batch[idx])
